# Initial kernel scaffold; baseline (speedup 1.0000x reference)
import jax
import jax.numpy as jnp
from jax.experimental import pallas as pl


def kernel(x, Wr, Wg, Wu, Wd):
    raise NotImplementedError("write your pallas kernel here")


# speedup vs baseline: 2.7533x; 2.7533x over previous
import functools

import jax
import jax.numpy as jnp
from jax import lax
from jax.experimental import pallas as pl
from jax.experimental.pallas import tpu as pltpu
from jax.experimental.pallas import tpu_sc as plsc

_NUM_EXPERTS = 16
_TOP_K = 2
_TM = 128

_SC_NC = 2
_SC_NS = 16
_SC_NW = _SC_NC * _SC_NS


def _sc_gather_rows(table, idx, chunk):
    n_rows, d = table.shape
    b = idx.shape[0]
    per_w = b // _SC_NW
    assert per_w % chunk == 0 and b % (8 * _SC_NW) == 0
    n_chunks = per_w // chunk
    mesh = plsc.VectorSubcoreMesh(core_axis_name="c", subcore_axis_name="s")

    @functools.partial(
        pl.kernel,
        mesh=mesh,
        out_type=jax.ShapeDtypeStruct((b, d), table.dtype),
        scratch_types=[
            pltpu.VMEM((per_w,), jnp.int32),
            pltpu.VMEM((chunk, d), table.dtype),
            pltpu.SemaphoreType.DMA,
        ],
    )
    def gather_kernel(table_hbm, idx_hbm, out_hbm, idx_v, rows_v, sem):
        wid = lax.axis_index("s") * _SC_NC + lax.axis_index("c")
        base = wid * per_w
        pltpu.sync_copy(idx_hbm.at[pl.ds(base, per_w)], idx_v)
        for j in range(n_chunks):
            pltpu.async_copy(
                table_hbm.at[idx_v.at[pl.ds(j * chunk, chunk)]], rows_v, sem
            ).wait()
            pltpu.sync_copy(rows_v, out_hbm.at[pl.ds(base + j * chunk, chunk)])

    return gather_kernel(table, idx)


def _sc_combine_pairs(table, idx_even, idx_odd, chunk):
    n_rows, d = table.shape
    t = idx_even.shape[0]
    per_w = t // _SC_NW
    assert per_w % chunk == 0 and t % (8 * _SC_NW) == 0
    n_chunks = per_w // chunk
    mesh = plsc.VectorSubcoreMesh(core_axis_name="c", subcore_axis_name="s")

    @functools.partial(
        pl.kernel,
        mesh=mesh,
        out_type=jax.ShapeDtypeStruct((t, d), table.dtype),
        scratch_types=[
            pltpu.VMEM((per_w,), jnp.int32),
            pltpu.VMEM((per_w,), jnp.int32),
            pltpu.VMEM((chunk, d), table.dtype),
            pltpu.VMEM((chunk, d), table.dtype),
            pltpu.SemaphoreType.DMA,
        ],
    )
    def combine_kernel(table_hbm, ie_hbm, io_hbm, out_hbm, ie_v, io_v, a_v, b_v, sem):
        wid = lax.axis_index("s") * _SC_NC + lax.axis_index("c")
        base = wid * per_w
        pltpu.sync_copy(ie_hbm.at[pl.ds(base, per_w)], ie_v)
        pltpu.sync_copy(io_hbm.at[pl.ds(base, per_w)], io_v)
        for c in range(n_chunks):
            pltpu.async_copy(
                table_hbm.at[ie_v.at[pl.ds(c * chunk, chunk)]], a_v, sem
            ).wait()
            pltpu.async_copy(
                table_hbm.at[io_v.at[pl.ds(c * chunk, chunk)]], b_v, sem
            ).wait()

            def row_add(r, _):
                for kk in range(d // 16):
                    sl = pl.ds(kk * 16, 16)
                    a_v[r, sl] = a_v[r, sl] + b_v[r, sl]
                return _

            lax.fori_loop(0, chunk, row_add, 0)
            pltpu.sync_copy(a_v, out_hbm.at[pl.ds(base + c * chunk, chunk)])

    return combine_kernel(table, idx_even, idx_odd)


def _gmm_body(te_ref, tv_ref, hs_ref, w_ref, wg_ref, wu_ref, wd_ref, out_ref):
    t = pl.program_id(0)

    @pl.when(tv_ref[t] == 1)
    def _():
        rows = hs_ref[...]
        wg = wg_ref[0]
        wu = wu_ref[0]
        wd = wd_ref[0]
        dn = (((1,), (1,)), ((), ()))
        g = jax.lax.dot_general(rows, wg, dn, preferred_element_type=jnp.float32)
        u = jax.lax.dot_general(rows, wu, dn, preferred_element_type=jnp.float32)
        a = (g * jax.nn.sigmoid(g)) * u
        d = jax.lax.dot_general(a, wd, dn, preferred_element_type=jnp.float32)
        out_ref[...] = d * w_ref[...]


def kernel(x, Wr, Wg, Wu, Wd):
    bsz, seq, dim = x.shape
    moe_dim = Wg.shape[1]
    h = x.reshape(-1, dim)
    T = h.shape[0]
    P = T * _TOP_K

    logits = h @ Wr.T
    scores = jax.nn.softmax(logits.astype(jnp.float32), axis=-1)
    topk_w, topk_idx = jax.lax.top_k(scores, _TOP_K)

    flat_e = topk_idx.reshape(-1).astype(jnp.int32)
    w_flat = topk_w.reshape(-1)
    onehot = (flat_e[:, None] == jnp.arange(_NUM_EXPERTS, dtype=jnp.int32)[None, :]
              ).astype(jnp.int32)
    cum_oh = jnp.cumsum(onehot, axis=0)
    counts = cum_oh[-1]
    rank = ((cum_oh - onehot) * onehot).sum(axis=1)

    padded = ((counts + _TM - 1) // _TM) * _TM
    pstart = (jnp.cumsum(padded) - padded).astype(jnp.int32)
    gstart = (jnp.cumsum(counts) - counts).astype(jnp.int32)
    dst = pstart[flat_e] + rank

    order = jnp.argsort(flat_e, stable=True)

    M_pad = P + _NUM_EXPERTS * _TM
    NT = M_pad // _TM
    slot = jnp.arange(M_pad, dtype=jnp.int32)
    cum_padded = jnp.cumsum(padded).astype(jnp.int32)
    slot_e = (slot[:, None] >= cum_padded[None, :]).astype(jnp.int32).sum(axis=1)
    slot_e = jnp.minimum(slot_e, _NUM_EXPERTS - 1)
    j = slot - pstart[slot_e]
    real = j < counts[slot_e]
    pair_idx = order[jnp.clip(gstart[slot_e] + jnp.minimum(j, counts[slot_e] - 1),
                              0, P - 1)]
    src = jnp.where(real, pair_idx // _TOP_K, slot % T).astype(jnp.int32)
    w_slot = jnp.where(real, w_flat[pair_idx], 0.0)

    hs_pad = _sc_gather_rows(h, src, 64)

    tiles = (padded // _TM).astype(jnp.int32)
    cum_tiles = jnp.cumsum(tiles)
    t_idx = jnp.arange(NT, dtype=jnp.int32)
    tile_e = jnp.searchsorted(cum_tiles, t_idx, side="right").astype(jnp.int32)
    tile_valid = (tile_e < _NUM_EXPERTS).astype(jnp.int32)
    tile_e = jnp.minimum(tile_e, _NUM_EXPERTS - 1)

    grid_spec = pltpu.PrefetchScalarGridSpec(
        num_scalar_prefetch=2,
        grid=(NT,),
        in_specs=[
            pl.BlockSpec((_TM, dim), lambda t, te, tv: (t, 0)),
            pl.BlockSpec((_TM, 1), lambda t, te, tv: (t, 0)),
            pl.BlockSpec((1, moe_dim, dim), lambda t, te, tv: (te[t], 0, 0)),
            pl.BlockSpec((1, moe_dim, dim), lambda t, te, tv: (te[t], 0, 0)),
            pl.BlockSpec((1, dim, moe_dim), lambda t, te, tv: (te[t], 0, 0)),
        ],
        out_specs=pl.BlockSpec((_TM, dim), lambda t, te, tv: (t, 0)),
    )
    out_pad = pl.pallas_call(
        _gmm_body,
        grid_spec=grid_spec,
        out_shape=jax.ShapeDtypeStruct((M_pad, dim), jnp.float32),
        compiler_params=pltpu.CompilerParams(
            dimension_semantics=("arbitrary",)),
    )(tile_e, tile_valid, hs_pad, w_slot[:, None], Wg, Wu, Wd)

    y = _sc_combine_pairs(out_pad, dst[0::2], dst[1::2], 32)
    return y.reshape(bsz, seq, dim)
